# Initial kernel scaffold; baseline (speedup 1.0000x reference)
import jax
import jax.numpy as jnp
from jax.experimental import pallas as pl


def kernel(x_nchw, wc, bc, wfc, bfc, mask):
    raise NotImplementedError("write your pallas kernel here")


# speedup vs baseline: 1.4920x; 1.4920x over previous
import jax
import jax.numpy as jnp
from jax.experimental import pallas as pl
from jax.experimental.pallas import tpu as pltpu


def _conv_body(L, C, SP, B, MG, PW):
    SPB = B * SP
    deltas = [(dy - 1) * PW + (dx - 1) for dy in range(3) for dx in range(3)]
    d8 = [d for t, d in enumerate(deltas) if t != 4]

    def body(x_ref, mask_ref, w8_ref, wc_ref, bc_ref, o_ref, scr_ref, x8_ref):
        scr_ref[:, :MG] = jnp.zeros((C, MG), jnp.bfloat16)
        scr_ref[:, MG + SPB:] = jnp.zeros((C, MG), jnp.bfloat16)
        scr_ref[:, MG:MG + SPB] = x_ref[0]
        mask = mask_ref[...]
        for l in range(L):
            for j, d in enumerate(d8):
                x8_ref[j * C:(j + 1) * C, :] = scr_ref[:, MG + d: MG + d + SPB]
            centre = scr_ref[:, MG:MG + SPB]
            z = jnp.dot(w8_ref[l], x8_ref[...],
                        preferred_element_type=jnp.float32)
            z = z + jnp.dot(wc_ref[l], centre,
                            preferred_element_type=jnp.float32)
            a = jnp.maximum(z + bc_ref[l], 0.0) * mask
            ab = a.astype(jnp.bfloat16)
            if l + 1 < L:
                scr_ref[:, MG:MG + SPB] = ab
            else:
                for b in range(B):
                    o_ref[0, b] = ab[:, b * SP:(b + 1) * SP]
    return body


def _fc_body(r_ref, w_ref, b_ref, o_ref):
    o_ref[...] = jnp.dot(r_ref[...], w_ref[...],
                         preferred_element_type=jnp.float32) + b_ref[...]


def kernel(x_nchw, wc, bc, wfc, bfc, mask):
    N, cin, H, W = x_nchw.shape
    L = wc.shape[0]
    C = wc.shape[2]
    NC = bfc.shape[1]
    SP = wfc.shape[2]
    PH, PW = H + 2, W + 2
    MG = 32
    assert MG >= PW + 1 and SP >= PH * PW and SP % 128 == 0 and cin <= C

    B = 32
    steps = -(-N // B)
    N_pad = steps * B
    SPB = B * SP

    xp = jnp.pad(x_nchw.astype(jnp.float32),
                 ((0, N_pad - N), (0, C - cin), (1, 1), (1, 1)))
    xp = xp.reshape(N_pad, C, PH * PW)
    xp = jnp.pad(xp, ((0, 0), (0, 0), (0, SP - PH * PW)))
    xp = xp.reshape(steps, B, C, SP).transpose(0, 2, 1, 3).reshape(steps, C, SPB)
    xp = xp.astype(jnp.bfloat16)
    mask_b = jnp.tile(mask, (1, B))

    w_all = jnp.transpose(wc, (0, 2, 1, 3))
    idx8 = [0, 1, 2, 3, 5, 6, 7, 8]
    w8 = w_all[:, :, idx8, :].reshape(L, C, 8 * C).astype(jnp.bfloat16)
    wcen = w_all[:, :, 4, :].astype(jnp.bfloat16)

    act = pl.pallas_call(
        _conv_body(L, C, SP, B, MG, PW),
        out_shape=jax.ShapeDtypeStruct((steps, B, C, SP), jnp.bfloat16),
        grid=(steps,),
        in_specs=[
            pl.BlockSpec((1, C, SPB), lambda s: (s, 0, 0)),
            pl.BlockSpec((1, SPB), lambda s: (0, 0)),
            pl.BlockSpec((L, C, 8 * C), lambda s: (0, 0, 0)),
            pl.BlockSpec((L, C, C), lambda s: (0, 0, 0)),
            pl.BlockSpec((L, C, 1), lambda s: (0, 0, 0)),
        ],
        out_specs=pl.BlockSpec((1, B, C, SP), lambda s: (s, 0, 0, 0)),
        scratch_shapes=[
            pltpu.VMEM((C, MG + SPB + MG), jnp.bfloat16),
            pltpu.VMEM((8 * C, SPB), jnp.bfloat16),
        ],
        compiler_params=pltpu.CompilerParams(
            dimension_semantics=("parallel",)),
    )(xp, mask_b, w8, wcen, bc)

    rows = act.reshape(N_pad, C * SP)
    wfct = jnp.transpose(wfc, (0, 2, 1)).reshape(C * SP, NC).astype(jnp.bfloat16)

    MB = 256 if N_pad % 256 == 0 else B
    fsteps = N_pad // MB
    scores = pl.pallas_call(
        _fc_body,
        out_shape=jax.ShapeDtypeStruct((N_pad, NC), jnp.float32),
        grid=(fsteps,),
        in_specs=[
            pl.BlockSpec((MB, C * SP), lambda s: (s, 0)),
            pl.BlockSpec((C * SP, NC), lambda s: (0, 0)),
            pl.BlockSpec((1, NC), lambda s: (0, 0)),
        ],
        out_specs=pl.BlockSpec((MB, NC), lambda s: (s, 0)),
        compiler_params=pltpu.CompilerParams(
            dimension_semantics=("parallel",)),
    )(rows, wfct, bfc)

    return scores[:N], None, None


# speedup vs baseline: 1.6164x; 1.0833x over previous
import jax
import jax.numpy as jnp
from jax.experimental import pallas as pl
from jax.experimental.pallas import tpu as pltpu


def _conv_body(L, C, SP, B, MG, PW):
    SPB = B * SP
    deltas = [(dy - 1) * PW + (dx - 1) for dy in range(3) for dx in range(3)]
    d8 = [d for t, d in enumerate(deltas) if t != 4]

    def body(x_ref, mask_ref, w8_ref, wc_ref, bc_ref, o_ref, scr_ref, x8_ref):
        CIN8 = x_ref.shape[1]
        scr_ref[...] = jnp.zeros(scr_ref.shape, jnp.bfloat16)
        scr_ref[:CIN8, MG:MG + SPB] = x_ref[0]
        mask = mask_ref[...]
        for l in range(L):
            for j, d in enumerate(d8):
                x8_ref[j * C:(j + 1) * C, :] = scr_ref[:, MG + d: MG + d + SPB]
            centre = scr_ref[:, MG:MG + SPB]
            z = jnp.dot(w8_ref[l], x8_ref[...],
                        preferred_element_type=jnp.float32)
            z = z + jnp.dot(wc_ref[l], centre,
                            preferred_element_type=jnp.float32)
            a = jnp.maximum(z + bc_ref[l], 0.0) * mask
            ab = a.astype(jnp.bfloat16)
            if l + 1 < L:
                scr_ref[:, MG:MG + SPB] = ab
            else:
                for b in range(B):
                    o_ref[0, b] = ab[:, b * SP:(b + 1) * SP]
    return body


def _fc_body(r_ref, w_ref, b_ref, o_ref):
    o_ref[...] = jnp.dot(r_ref[...], w_ref[...],
                         preferred_element_type=jnp.float32) + b_ref[...]


def kernel(x_nchw, wc, bc, wfc, bfc, mask):
    N, cin, H, W = x_nchw.shape
    L = wc.shape[0]
    C = wc.shape[2]
    NC = bfc.shape[1]
    SP = wfc.shape[2]
    PH, PW = H + 2, W + 2
    MG = 32
    assert MG >= PW + 1 and SP >= PH * PW and SP % 128 == 0 and cin <= C

    B = 32
    steps = -(-N // B)
    N_pad = steps * B
    SPB = B * SP

    CIN8 = min(C, ((cin + 7) // 8) * 8)
    xp = jnp.pad(x_nchw.astype(jnp.bfloat16),
                 ((0, N_pad - N), (0, CIN8 - cin), (1, 1), (1, 1)))
    xp = xp.reshape(N_pad, CIN8, PH * PW)
    xp = jnp.pad(xp, ((0, 0), (0, 0), (0, SP - PH * PW)))
    xp = xp.reshape(steps, B, CIN8, SP).transpose(0, 2, 1, 3).reshape(
        steps, CIN8, SPB)
    mask_b = jnp.tile(mask, (1, B))

    w_all = jnp.transpose(wc, (0, 2, 1, 3))
    idx8 = [0, 1, 2, 3, 5, 6, 7, 8]
    w8 = w_all[:, :, idx8, :].reshape(L, C, 8 * C).astype(jnp.bfloat16)
    wcen = w_all[:, :, 4, :].astype(jnp.bfloat16)

    act = pl.pallas_call(
        _conv_body(L, C, SP, B, MG, PW),
        out_shape=jax.ShapeDtypeStruct((steps, B, C, SP), jnp.bfloat16),
        grid=(steps,),
        in_specs=[
            pl.BlockSpec((1, CIN8, SPB), lambda s: (s, 0, 0)),
            pl.BlockSpec((1, SPB), lambda s: (0, 0)),
            pl.BlockSpec((L, C, 8 * C), lambda s: (0, 0, 0)),
            pl.BlockSpec((L, C, C), lambda s: (0, 0, 0)),
            pl.BlockSpec((L, C, 1), lambda s: (0, 0, 0)),
        ],
        out_specs=pl.BlockSpec((1, B, C, SP), lambda s: (s, 0, 0, 0)),
        scratch_shapes=[
            pltpu.VMEM((C, MG + SPB + MG), jnp.bfloat16),
            pltpu.VMEM((8 * C, SPB), jnp.bfloat16),
        ],
        compiler_params=pltpu.CompilerParams(
            dimension_semantics=("parallel",)),
    )(xp, mask_b, w8, wcen, bc)

    rows = act.reshape(N_pad, C * SP)
    wfct = jnp.transpose(wfc.astype(jnp.bfloat16), (0, 2, 1)).reshape(C * SP, NC)

    MB = 256 if N_pad % 256 == 0 else B
    fsteps = N_pad // MB
    scores = pl.pallas_call(
        _fc_body,
        out_shape=jax.ShapeDtypeStruct((N_pad, NC), jnp.float32),
        grid=(fsteps,),
        in_specs=[
            pl.BlockSpec((MB, C * SP), lambda s: (s, 0)),
            pl.BlockSpec((C * SP, NC), lambda s: (0, 0)),
            pl.BlockSpec((1, NC), lambda s: (0, 0)),
        ],
        out_specs=pl.BlockSpec((MB, NC), lambda s: (s, 0)),
        compiler_params=pltpu.CompilerParams(
            dimension_semantics=("parallel",)),
    )(rows, wfct, bfc)

    return scores[:N], None, None


# speedup vs baseline: 1.6357x; 1.0120x over previous
import jax
import jax.numpy as jnp
from jax.experimental import pallas as pl
from jax.experimental.pallas import tpu as pltpu


def _conv_body(L, C, SP, B, MG, PW):
    SPB = B * SP
    deltas = [(dy - 1) * PW + (dx - 1) for dy in range(3) for dx in range(3)]
    d8 = [d for t, d in enumerate(deltas) if t != 4]

    def body(x_ref, mask_ref, w8_ref, wc_ref, bc_ref, o_ref, scr_ref, x8_ref):
        CIN8 = x_ref.shape[2]
        scr_ref[...] = jnp.zeros(scr_ref.shape, jnp.bfloat16)
        for b in range(B):
            scr_ref[:CIN8, MG + b * SP:MG + (b + 1) * SP] = x_ref[0, b]
        mask = mask_ref[...]
        for l in range(L):
            for j, d in enumerate(d8):
                x8_ref[j * C:(j + 1) * C, :] = scr_ref[:, MG + d: MG + d + SPB]
            centre = scr_ref[:, MG:MG + SPB]
            z = jnp.dot(w8_ref[l], x8_ref[...],
                        preferred_element_type=jnp.float32)
            z = z + jnp.dot(wc_ref[l], centre,
                            preferred_element_type=jnp.float32)
            a = jnp.maximum(z + bc_ref[l], 0.0) * mask
            ab = a.astype(jnp.bfloat16)
            if l + 1 < L:
                scr_ref[:, MG:MG + SPB] = ab
            else:
                for b in range(B):
                    o_ref[0, b] = ab[:, b * SP:(b + 1) * SP]
    return body


def _fc_body(C, SP):
    def fc(r_ref, w_ref, b_ref, o_ref):
        acc = None
        for c in range(C):
            p = jax.lax.dot_general(
                r_ref[:, c * SP:(c + 1) * SP], w_ref[c],
                (((1,), (1,)), ((), ())),
                preferred_element_type=jnp.float32)
            acc = p if acc is None else acc + p
        o_ref[...] = acc + b_ref[...]
    return fc


def kernel(x_nchw, wc, bc, wfc, bfc, mask):
    N, cin, H, W = x_nchw.shape
    L = wc.shape[0]
    C = wc.shape[2]
    NC = bfc.shape[1]
    SP = wfc.shape[2]
    PH, PW = H + 2, W + 2
    MG = 32
    assert MG >= PW + 1 and SP >= PH * PW and SP % 128 == 0 and cin <= C

    B = 32
    steps = -(-N // B)
    N_pad = steps * B
    SPB = B * SP

    CIN8 = min(C, ((cin + 7) // 8) * 8)
    xp = jnp.pad(x_nchw.astype(jnp.bfloat16),
                 ((0, N_pad - N), (0, CIN8 - cin), (1, 1), (1, 1)))
    xp = xp.reshape(N_pad, CIN8, PH * PW)
    xp = jnp.pad(xp, ((0, 0), (0, 0), (0, SP - PH * PW)))
    xp = xp.reshape(steps, B, CIN8, SP)
    mask_b = jnp.tile(mask, (1, B))

    w_all = jnp.transpose(wc, (0, 2, 1, 3))
    idx8 = [0, 1, 2, 3, 5, 6, 7, 8]
    w8 = w_all[:, :, idx8, :].reshape(L, C, 8 * C).astype(jnp.bfloat16)
    wcen = w_all[:, :, 4, :].astype(jnp.bfloat16)

    act = pl.pallas_call(
        _conv_body(L, C, SP, B, MG, PW),
        out_shape=jax.ShapeDtypeStruct((steps, B, C, SP), jnp.bfloat16),
        grid=(steps,),
        in_specs=[
            pl.BlockSpec((1, B, CIN8, SP), lambda s: (s, 0, 0, 0)),
            pl.BlockSpec((1, SPB), lambda s: (0, 0)),
            pl.BlockSpec((L, C, 8 * C), lambda s: (0, 0, 0)),
            pl.BlockSpec((L, C, C), lambda s: (0, 0, 0)),
            pl.BlockSpec((L, C, 1), lambda s: (0, 0, 0)),
        ],
        out_specs=pl.BlockSpec((1, B, C, SP), lambda s: (s, 0, 0, 0)),
        scratch_shapes=[
            pltpu.VMEM((C, MG + SPB + MG), jnp.bfloat16),
            pltpu.VMEM((8 * C, SPB), jnp.bfloat16),
        ],
        compiler_params=pltpu.CompilerParams(
            dimension_semantics=("parallel",)),
    )(xp, mask_b, w8, wcen, bc)

    rows = act.reshape(N_pad, C * SP)
    wfcb = wfc.astype(jnp.bfloat16)

    MB = 256 if N_pad % 256 == 0 else B
    fsteps = N_pad // MB
    scores = pl.pallas_call(
        _fc_body(C, SP),
        out_shape=jax.ShapeDtypeStruct((N_pad, NC), jnp.float32),
        grid=(fsteps,),
        in_specs=[
            pl.BlockSpec((MB, C * SP), lambda s: (s, 0)),
            pl.BlockSpec((C, NC, SP), lambda s: (0, 0, 0)),
            pl.BlockSpec((1, NC), lambda s: (0, 0)),
        ],
        out_specs=pl.BlockSpec((MB, NC), lambda s: (s, 0)),
        compiler_params=pltpu.CompilerParams(
            dimension_semantics=("parallel",)),
    )(rows, wfcb, bfc)

    return scores[:N], None, None


# speedup vs baseline: 2.5155x; 1.5378x over previous
import jax
import jax.numpy as jnp
from jax.experimental import pallas as pl
from jax.experimental.pallas import tpu as pltpu


def _conv_body(L, C, SP, B, PW):
    SPB = B * SP
    deltas = [(dy - 1) * PW + (dx - 1) for dy in range(3) for dx in range(3)]
    d4a = [deltas[t] for t in (0, 1, 2, 3)]
    d4b = [deltas[t] for t in (5, 6, 7, 8)]

    def body(x_ref, mask_ref, w4a_ref, w4b_ref, wc_ref, bc_ref, o_ref,
             scr_ref, xa_ref, xb_ref):
        CIN8 = x_ref.shape[2]
        scr_ref[CIN8:, :] = jnp.zeros((C - CIN8, SPB), jnp.bfloat16)
        for b in range(B):
            scr_ref[:CIN8, b * SP:(b + 1) * SP] = x_ref[0, b]
        mask = mask_ref[...]
        for l in range(L):
            av = scr_ref[...]
            for j, d in enumerate(d4a):
                xa_ref[j * C:(j + 1) * C, :] = jnp.roll(av, -d, axis=1)
            for j, d in enumerate(d4b):
                xb_ref[j * C:(j + 1) * C, :] = jnp.roll(av, -d, axis=1)
            z = jnp.dot(w4a_ref[l], xa_ref[...],
                        preferred_element_type=jnp.float32)
            z = z + jnp.dot(w4b_ref[l], xb_ref[...],
                            preferred_element_type=jnp.float32)
            z = z + jnp.dot(wc_ref[l], av,
                            preferred_element_type=jnp.float32)
            ab = jnp.maximum(z + bc_ref[l], 0.0).astype(jnp.bfloat16) * mask
            if l + 1 < L:
                scr_ref[...] = ab
            else:
                for b in range(B):
                    o_ref[0, b] = ab[:, b * SP:(b + 1) * SP]
    return body


def _fc_body(C, SP):
    def fc(r_ref, w_ref, b_ref, o_ref):
        acc = None
        for c in range(C):
            p = jax.lax.dot_general(
                r_ref[:, c * SP:(c + 1) * SP], w_ref[c],
                (((1,), (1,)), ((), ())),
                preferred_element_type=jnp.float32)
            acc = p if acc is None else acc + p
        o_ref[...] = acc + b_ref[...]
    return fc


def kernel(x_nchw, wc, bc, wfc, bfc, mask):
    N, cin, H, W = x_nchw.shape
    L = wc.shape[0]
    C = wc.shape[2]
    NC = bfc.shape[1]
    SP = wfc.shape[2]
    PH, PW = H + 2, W + 2
    assert SP >= PH * PW and SP % 128 == 0 and cin <= C
    assert SP - (PH - 1) * PW - (PW - 1) > PW + 1 > 0

    B = 32
    steps = -(-N // B)
    N_pad = steps * B
    SPB = B * SP

    CIN8 = min(C, ((cin + 7) // 8) * 8)
    xp = jnp.pad(x_nchw.astype(jnp.bfloat16),
                 ((0, N_pad - N), (0, CIN8 - cin), (1, 1), (1, 1)))
    xp = xp.reshape(N_pad, CIN8, PH * PW)
    xp = jnp.pad(xp, ((0, 0), (0, 0), (0, SP - PH * PW)))
    xp = xp.reshape(steps, B, CIN8, SP)
    mask_b = jnp.tile(mask, (1, B)).astype(jnp.bfloat16)

    w_all = jnp.transpose(wc, (0, 2, 1, 3))
    w4a = w_all[:, :, [0, 1, 2, 3], :].reshape(L, C, 4 * C).astype(jnp.bfloat16)
    w4b = w_all[:, :, [5, 6, 7, 8], :].reshape(L, C, 4 * C).astype(jnp.bfloat16)
    wcen = w_all[:, :, 4, :].astype(jnp.bfloat16)

    act = pl.pallas_call(
        _conv_body(L, C, SP, B, PW),
        out_shape=jax.ShapeDtypeStruct((steps, B, C, SP), jnp.bfloat16),
        grid=(steps,),
        in_specs=[
            pl.BlockSpec((1, B, CIN8, SP), lambda s: (s, 0, 0, 0)),
            pl.BlockSpec((1, SPB), lambda s: (0, 0)),
            pl.BlockSpec((L, C, 4 * C), lambda s: (0, 0, 0)),
            pl.BlockSpec((L, C, 4 * C), lambda s: (0, 0, 0)),
            pl.BlockSpec((L, C, C), lambda s: (0, 0, 0)),
            pl.BlockSpec((L, C, 1), lambda s: (0, 0, 0)),
        ],
        out_specs=pl.BlockSpec((1, B, C, SP), lambda s: (s, 0, 0, 0)),
        scratch_shapes=[
            pltpu.VMEM((C, SPB), jnp.bfloat16),
            pltpu.VMEM((4 * C, SPB), jnp.bfloat16),
            pltpu.VMEM((4 * C, SPB), jnp.bfloat16),
        ],
        compiler_params=pltpu.CompilerParams(
            dimension_semantics=("parallel",)),
    )(xp, mask_b, w4a, w4b, wcen, bc)

    rows = act.reshape(N_pad, C * SP)
    wfcb = wfc.astype(jnp.bfloat16)

    MB = 256 if N_pad % 256 == 0 else B
    fsteps = N_pad // MB
    scores = pl.pallas_call(
        _fc_body(C, SP),
        out_shape=jax.ShapeDtypeStruct((N_pad, NC), jnp.float32),
        grid=(fsteps,),
        in_specs=[
            pl.BlockSpec((MB, C * SP), lambda s: (s, 0)),
            pl.BlockSpec((C, NC, SP), lambda s: (0, 0, 0)),
            pl.BlockSpec((1, NC), lambda s: (0, 0)),
        ],
        out_specs=pl.BlockSpec((MB, NC), lambda s: (s, 0)),
        compiler_params=pltpu.CompilerParams(
            dimension_semantics=("parallel",)),
    )(rows, wfcb, bfc)

    return scores[:N], None, None
